# baseline (speedup 1.0000x reference)
import jax
import jax.numpy as jnp
from jax.experimental import pallas as pl

NSTACK = 3
NNODE = 25
TDIM = 300
TPAD = 304
TGRP = 8
RCHUNK = TGRP * NNODE
TTILE = 152
RB = TTILE * NNODE
NCH = RB // RCHUNK
HID = 256


def _gin_body(f_ref, bd_ref, w0_ref, b0_ref, w1_ref, b1_ref,
              w2_ref, b2_ref, out_ref):
    j = pl.program_id(1)
    rows = jax.lax.broadcasted_iota(jnp.int32, (RB, 1), 0)
    limit = jnp.where(j == TPAD // TTILE - 1, RB - (TPAD - TDIM) * NNODE, RB)
    f = jnp.where(rows < limit, f_ref[0], 0.0)
    bd = bd_ref[...]

    bdh = bd.astype(jnp.bfloat16)

    def bd_apply(x):
        x16 = x.astype(jnp.bfloat16)
        return jnp.concatenate(
            [jnp.dot(bdh, x16[k * RCHUNK:(k + 1) * RCHUNK, :],
                     preferred_element_type=jnp.float32)
             for k in range(NCH)], axis=0)

    agg0 = bd_apply(f)
    acc = None
    for s in range(NSTACK):
        h = jnp.maximum(
            jnp.dot(agg0.astype(jnp.bfloat16),
                    w0_ref[s].astype(jnp.bfloat16),
                    preferred_element_type=jnp.float32)
            + b0_ref[s:s + 1, :], 0.0)
        h = jnp.maximum(
            jnp.dot(bd_apply(h).astype(jnp.bfloat16),
                    w1_ref[s].astype(jnp.bfloat16),
                    preferred_element_type=jnp.float32)
            + b1_ref[s:s + 1, :], 0.0)
        g = jnp.dot(h.astype(jnp.bfloat16), w2_ref[s].astype(jnp.bfloat16),
                    preferred_element_type=jnp.float32)
        o = jnp.maximum(bd_apply(g) + b2_ref[s:s + 1, :], 0.0)
        acc = o if acc is None else acc + o
    out_ref[0] = acc * (1.0 / NSTACK)


def _mlp_body(x_ref, wf0_ref, bf0_ref, wf1_ref, bf1_ref, out_ref):
    hfc = jnp.maximum(
        jnp.dot(x_ref[...], wf0_ref[...], preferred_element_type=jnp.float32)
        + bf0_ref[...], 0.0)
    out_ref[...] = (jnp.dot(hfc, wf1_ref[...],
                            preferred_element_type=jnp.float32)
                    + bf1_ref[...])


def kernel(features, A, W0, b0, W1, b1, W2, b2, Wf0, bf0, Wf1, bf1):
    B = features.shape[0]
    m_hat = A + jnp.eye(NNODE, dtype=A.dtype)
    bd = jnp.kron(jnp.eye(TGRP, dtype=A.dtype), m_hat)

    f2 = features.reshape(B, TDIM * NNODE, 3)

    gin = pl.pallas_call(
        _gin_body,
        grid=(B, TPAD // TTILE),
        in_specs=[
            pl.BlockSpec((1, RB, 3), lambda b, j: (b, j, 0)),
            pl.BlockSpec((RCHUNK, RCHUNK), lambda b, j: (0, 0)),
            pl.BlockSpec((NSTACK, 3, HID), lambda b, j: (0, 0, 0)),
            pl.BlockSpec((NSTACK, HID), lambda b, j: (0, 0)),
            pl.BlockSpec((NSTACK, HID, HID), lambda b, j: (0, 0, 0)),
            pl.BlockSpec((NSTACK, HID), lambda b, j: (0, 0)),
            pl.BlockSpec((NSTACK, HID, 1), lambda b, j: (0, 0, 0)),
            pl.BlockSpec((NSTACK, 1), lambda b, j: (0, 0)),
        ],
        out_specs=pl.BlockSpec((1, RB, 1), lambda b, j: (b, j, 0)),
        out_shape=jax.ShapeDtypeStruct((B, TDIM * NNODE, 1), jnp.float32),
    )(f2, bd, W0, b0, W1, b1, W2, b2)

    pooled = gin.reshape(B, TDIM * NNODE)
    logits = pl.pallas_call(
        _mlp_body,
        out_shape=jax.ShapeDtypeStruct((B, 60), jnp.float32),
    )(pooled, Wf0, bf0.reshape(1, -1), Wf1, bf1.reshape(1, -1))
    return logits


# speedup vs baseline: 1.0959x; 1.0959x over previous
import jax
import jax.numpy as jnp
from jax.experimental import pallas as pl

NSTACK = 3
NNODE = 25
TDIM = 300
TPAD = 304
TGRP = 8
RCHUNK = TGRP * NNODE
TTILE = 304
RB = TTILE * NNODE
NCH = RB // RCHUNK
HID = 256


def _gin_body(f_ref, bd_ref, w0_ref, b0_ref, w1_ref, b1_ref,
              w2_ref, b2_ref, out_ref):
    j = pl.program_id(1)
    rows = jax.lax.broadcasted_iota(jnp.int32, (RB, 1), 0)
    limit = jnp.where(j == TPAD // TTILE - 1, RB - (TPAD - TDIM) * NNODE, RB)
    f = jnp.where(rows < limit, f_ref[0], 0.0)
    bd = bd_ref[...]

    def bd_apply(x):
        return jnp.concatenate(
            [jnp.dot(bd, x[k * RCHUNK:(k + 1) * RCHUNK, :],
                     preferred_element_type=jnp.float32)
             for k in range(NCH)], axis=0)

    agg0 = bd_apply(f)
    acc = None
    for s in range(NSTACK):
        h = jnp.maximum(
            jnp.dot(agg0, w0_ref[s], preferred_element_type=jnp.float32)
            + b0_ref[s:s + 1, :], 0.0)
        h = jnp.maximum(
            jnp.dot(bd_apply(h), w1_ref[s], preferred_element_type=jnp.float32)
            + b1_ref[s:s + 1, :], 0.0)
        g = jnp.dot(h, w2_ref[s], preferred_element_type=jnp.float32)
        o = jnp.maximum(bd_apply(g) + b2_ref[s:s + 1, :], 0.0)
        acc = o if acc is None else acc + o
    out_ref[0] = acc * (1.0 / NSTACK)


def _mlp_body(x_ref, wf0_ref, bf0_ref, wf1_ref, bf1_ref, out_ref):
    hfc = jnp.maximum(
        jnp.dot(x_ref[...], wf0_ref[...], preferred_element_type=jnp.float32)
        + bf0_ref[...], 0.0)
    out_ref[...] = (jnp.dot(hfc, wf1_ref[...],
                            preferred_element_type=jnp.float32)
                    + bf1_ref[...])


def kernel(features, A, W0, b0, W1, b1, W2, b2, Wf0, bf0, Wf1, bf1):
    B = features.shape[0]
    m_hat = A + jnp.eye(NNODE, dtype=A.dtype)
    bd = jnp.kron(jnp.eye(TGRP, dtype=A.dtype), m_hat)

    f2 = features.reshape(B, TDIM * NNODE, 3)

    gin = pl.pallas_call(
        _gin_body,
        grid=(B, TPAD // TTILE),
        in_specs=[
            pl.BlockSpec((1, RB, 3), lambda b, j: (b, j, 0)),
            pl.BlockSpec((RCHUNK, RCHUNK), lambda b, j: (0, 0)),
            pl.BlockSpec((NSTACK, 3, HID), lambda b, j: (0, 0, 0)),
            pl.BlockSpec((NSTACK, HID), lambda b, j: (0, 0)),
            pl.BlockSpec((NSTACK, HID, HID), lambda b, j: (0, 0, 0)),
            pl.BlockSpec((NSTACK, HID), lambda b, j: (0, 0)),
            pl.BlockSpec((NSTACK, HID, 1), lambda b, j: (0, 0, 0)),
            pl.BlockSpec((NSTACK, 1), lambda b, j: (0, 0)),
        ],
        out_specs=pl.BlockSpec((1, RB, 1), lambda b, j: (b, j, 0)),
        out_shape=jax.ShapeDtypeStruct((B, TDIM * NNODE, 1), jnp.float32),
    )(f2, bd, W0, b0, W1, b1, W2, b2)

    pooled = gin.reshape(B, TDIM * NNODE)
    logits = pl.pallas_call(
        _mlp_body,
        out_shape=jax.ShapeDtypeStruct((B, 60), jnp.float32),
    )(pooled, Wf0, bf0.reshape(1, -1), Wf1, bf1.reshape(1, -1))
    return logits


# speedup vs baseline: 1.0972x; 1.0012x over previous
import jax
import jax.numpy as jnp
from jax.experimental import pallas as pl

NSTACK = 3
NNODE = 25
TDIM = 300
TPAD = 304
TGRP = 8
RCHUNK = TGRP * NNODE
TTILE = 304
RB = TTILE * NNODE
NCH = RB // RCHUNK
HID = 256


def _gin_body(f_ref, bd_ref, w0_ref, b0_ref, w1_ref, b1_ref,
              w2_ref, b2_ref, out_ref):
    j = pl.program_id(1)
    rows = jax.lax.broadcasted_iota(jnp.int32, (RB, 1), 0)
    limit = jnp.where(j == TPAD // TTILE - 1, RB - (TPAD - TDIM) * NNODE, RB)
    f = jnp.where(rows < limit, f_ref[0], 0.0)
    bd = bd_ref[...]

    def bd_apply(x):
        return jnp.concatenate(
            [jnp.dot(bd, x[k * RCHUNK:(k + 1) * RCHUNK, :],
                     preferred_element_type=jnp.float32)
             for k in range(NCH)], axis=0)

    agg0 = bd_apply(f)
    acc = None
    for s in range(NSTACK):
        h = jnp.maximum(
            jnp.dot(agg0, w0_ref[s], preferred_element_type=jnp.float32)
            + b0_ref[s:s + 1, :], 0.0)
        h = jnp.maximum(
            jnp.dot(bd_apply(h), w1_ref[s], preferred_element_type=jnp.float32)
            + b1_ref[s:s + 1, :], 0.0)
        g = jnp.dot(h, w2_ref[s], preferred_element_type=jnp.float32)
        o = jnp.maximum(bd_apply(g) + b2_ref[s:s + 1, :], 0.0)
        acc = o if acc is None else acc + o
    out_ref[0] = acc * (1.0 / NSTACK)


def _mlp_body(x_ref, wf0_ref, bf0_ref, wf1_ref, bf1_ref, out_ref):
    hfc = jnp.maximum(
        jnp.dot(x_ref[...], wf0_ref[...], preferred_element_type=jnp.float32)
        + bf0_ref[...], 0.0)
    out_ref[...] = (jnp.dot(hfc, wf1_ref[...],
                            preferred_element_type=jnp.float32)
                    + bf1_ref[...])


def kernel(features, A, W0, b0, W1, b1, W2, b2, Wf0, bf0, Wf1, bf1):
    B = features.shape[0]
    m_hat = A + jnp.eye(NNODE, dtype=A.dtype)
    eye8 = jnp.eye(TGRP, dtype=A.dtype)
    bd = (eye8[:, None, :, None] * m_hat[None, :, None, :]).reshape(
        RCHUNK, RCHUNK)

    f2 = features.reshape(B, TDIM * NNODE, 3)

    gin = pl.pallas_call(
        _gin_body,
        grid=(B, TPAD // TTILE),
        in_specs=[
            pl.BlockSpec((1, RB, 3), lambda b, j: (b, j, 0)),
            pl.BlockSpec((RCHUNK, RCHUNK), lambda b, j: (0, 0)),
            pl.BlockSpec((NSTACK, 3, HID), lambda b, j: (0, 0, 0)),
            pl.BlockSpec((NSTACK, HID), lambda b, j: (0, 0)),
            pl.BlockSpec((NSTACK, HID, HID), lambda b, j: (0, 0, 0)),
            pl.BlockSpec((NSTACK, HID), lambda b, j: (0, 0)),
            pl.BlockSpec((NSTACK, HID, 1), lambda b, j: (0, 0, 0)),
            pl.BlockSpec((NSTACK, 1), lambda b, j: (0, 0)),
        ],
        out_specs=pl.BlockSpec((1, RB, 1), lambda b, j: (b, j, 0)),
        out_shape=jax.ShapeDtypeStruct((B, TDIM * NNODE, 1), jnp.float32),
    )(f2, bd, W0, b0, W1, b1, W2, b2)

    pooled = gin.reshape(B, TDIM * NNODE)
    logits = pl.pallas_call(
        _mlp_body,
        out_shape=jax.ShapeDtypeStruct((B, 60), jnp.float32),
    )(pooled, Wf0, bf0.reshape(1, -1), Wf1, bf1.reshape(1, -1))
    return logits


# speedup vs baseline: 1.1626x; 1.0595x over previous
import jax
import jax.numpy as jnp
from jax.experimental import pallas as pl

NSTACK = 3
NNODE = 25
TDIM = 300
TPAD = 304
TGRP = 8
RCHUNK = TGRP * NNODE
TTILE = 304
RB = TTILE * NNODE
NCH = RB // RCHUNK
HID = 256


def _gin_body(f_ref, bd_ref, w0_ref, b0_ref, w1_ref, b1_ref,
              w2_ref, b2_ref, out_ref):
    j = pl.program_id(1)
    rows = jax.lax.broadcasted_iota(jnp.int32, (RB, 1), 0)
    limit = jnp.where(j == TPAD // TTILE - 1, RB - (TPAD - TDIM) * NNODE, RB)
    f = jnp.where(rows < limit, f_ref[0], 0.0)
    bd = bd_ref[...]

    def bd_apply(x):
        return jnp.concatenate(
            [jnp.dot(bd, x[k * RCHUNK:(k + 1) * RCHUNK, :],
                     preferred_element_type=jnp.float32)
             for k in range(NCH)], axis=0)

    agg0 = bd_apply(f)
    acc = None
    for s in range(NSTACK):
        h = jnp.maximum(
            jnp.dot(agg0, w0_ref[s], preferred_element_type=jnp.float32)
            + b0_ref[s:s + 1, :], 0.0)
        h = jnp.maximum(
            jnp.dot(bd_apply(h), w1_ref[s], preferred_element_type=jnp.float32)
            + b1_ref[s:s + 1, :], 0.0)
        g = jnp.dot(h, w2_ref[s], preferred_element_type=jnp.float32)
        o = jnp.maximum(bd_apply(g) + b2_ref[s:s + 1, :], 0.0)
        acc = o if acc is None else acc + o
    row = jnp.transpose(acc * (1.0 / NSTACK))
    out_ref[pl.ds(pl.program_id(0), 1), :] = row[:, :TDIM * NNODE]


def _mlp_body(x_ref, wf0_ref, bf0_ref, wf1_ref, bf1_ref, out_ref):
    hfc = jnp.maximum(
        jnp.dot(x_ref[...], wf0_ref[...], preferred_element_type=jnp.float32)
        + bf0_ref[...], 0.0)
    out_ref[...] = (jnp.dot(hfc, wf1_ref[...],
                            preferred_element_type=jnp.float32)
                    + bf1_ref[...])


def kernel(features, A, W0, b0, W1, b1, W2, b2, Wf0, bf0, Wf1, bf1):
    B = features.shape[0]
    m_hat = A + jnp.eye(NNODE, dtype=A.dtype)
    eye8 = jnp.eye(TGRP, dtype=A.dtype)
    bd = (eye8[:, None, :, None] * m_hat[None, :, None, :]).reshape(
        RCHUNK, RCHUNK)

    f2 = features.reshape(B, TDIM * NNODE, 3)

    gin = pl.pallas_call(
        _gin_body,
        grid=(B, TPAD // TTILE),
        in_specs=[
            pl.BlockSpec((1, RB, 3), lambda b, j: (b, j, 0)),
            pl.BlockSpec((RCHUNK, RCHUNK), lambda b, j: (0, 0)),
            pl.BlockSpec((NSTACK, 3, HID), lambda b, j: (0, 0, 0)),
            pl.BlockSpec((NSTACK, HID), lambda b, j: (0, 0)),
            pl.BlockSpec((NSTACK, HID, HID), lambda b, j: (0, 0, 0)),
            pl.BlockSpec((NSTACK, HID), lambda b, j: (0, 0)),
            pl.BlockSpec((NSTACK, HID, 1), lambda b, j: (0, 0, 0)),
            pl.BlockSpec((NSTACK, 1), lambda b, j: (0, 0)),
        ],
        out_specs=pl.BlockSpec((B, TDIM * NNODE), lambda b, j: (0, 0)),
        out_shape=jax.ShapeDtypeStruct((B, TDIM * NNODE), jnp.float32),
    )(f2, bd, W0, b0, W1, b1, W2, b2)

    pooled = gin
    logits = pl.pallas_call(
        _mlp_body,
        out_shape=jax.ShapeDtypeStruct((B, 60), jnp.float32),
    )(pooled, Wf0, bf0.reshape(1, -1), Wf1, bf1.reshape(1, -1))
    return logits


# speedup vs baseline: 1.2748x; 1.0965x over previous
import jax
import jax.numpy as jnp
from jax.experimental import pallas as pl

NSTACK = 3
NNODE = 25
TDIM = 300
TPAD = 304
TGRP = 8
RCHUNK = TGRP * NNODE
TTILE = 304
RB = TTILE * NNODE
NCH = RB // RCHUNK
FBLK = 7552
HID = 256


def _tr_body(x_ref, out_ref):
    out_ref[0, :, 0, :TDIM * NNODE] = jnp.transpose(x_ref[0])


def _gin_body(f_ref, bd_ref, w0_ref, b0_ref, w1_ref, b1_ref,
              w2_ref, b2_ref, out_ref):
    fcb = jnp.transpose(f_ref[:, 0, 0, :])
    fcb = jnp.concatenate(
        [fcb, jnp.zeros((RB - FBLK, 3), jnp.float32)], axis=0)
    rows = jax.lax.broadcasted_iota(jnp.int32, (RB, 1), 0)
    f = jnp.where(rows < TDIM * NNODE, fcb, 0.0)
    bd = bd_ref[...]

    def bd_apply(x):
        return jnp.concatenate(
            [jnp.dot(bd, x[k * RCHUNK:(k + 1) * RCHUNK, :],
                     preferred_element_type=jnp.float32)
             for k in range(NCH)], axis=0)

    agg0 = bd_apply(f)
    acc = None
    for s in range(NSTACK):
        h = jnp.maximum(
            jnp.dot(agg0, w0_ref[s], preferred_element_type=jnp.float32)
            + b0_ref[s:s + 1, :], 0.0)
        h = jnp.maximum(
            jnp.dot(bd_apply(h), w1_ref[s], preferred_element_type=jnp.float32)
            + b1_ref[s:s + 1, :], 0.0)
        g = jnp.dot(h, w2_ref[s], preferred_element_type=jnp.float32)
        o = jnp.maximum(bd_apply(g) + b2_ref[s:s + 1, :], 0.0)
        acc = o if acc is None else acc + o
    row = jnp.transpose(acc * (1.0 / NSTACK))
    out_ref[pl.ds(pl.program_id(0), 1), :] = row[:, :TDIM * NNODE]


def _mlp_body(x_ref, wf0_ref, bf0_ref, wf1_ref, bf1_ref, out_ref):
    hfc = jnp.maximum(
        jnp.dot(x_ref[...], wf0_ref[...], preferred_element_type=jnp.float32)
        + bf0_ref[...], 0.0)
    out_ref[...] = (jnp.dot(hfc, wf1_ref[...],
                            preferred_element_type=jnp.float32)
                    + bf1_ref[...])


def kernel(features, A, W0, b0, W1, b1, W2, b2, Wf0, bf0, Wf1, bf1):
    B = features.shape[0]
    m_hat = A + jnp.eye(NNODE, dtype=A.dtype)
    eye8 = jnp.eye(TGRP, dtype=A.dtype)
    bd = (eye8[:, None, :, None] * m_hat[None, :, None, :]).reshape(
        RCHUNK, RCHUNK)

    ftr = features.transpose(3, 1, 2, 0).reshape(3, TDIM * NNODE, B)
    ft = pl.pallas_call(
        _tr_body,
        grid=(3,),
        in_specs=[pl.BlockSpec((1, TDIM * NNODE, B), lambda c: (c, 0, 0))],
        out_specs=pl.BlockSpec((1, B, 1, FBLK), lambda c: (c, 0, 0, 0)),
        out_shape=jax.ShapeDtypeStruct((3, B, 1, FBLK), jnp.float32),
    )(ftr)

    gin = pl.pallas_call(
        _gin_body,
        grid=(B, TPAD // TTILE),
        in_specs=[
            pl.BlockSpec((3, 1, 1, FBLK), lambda b, j: (0, b, 0, 0)),
            pl.BlockSpec((RCHUNK, RCHUNK), lambda b, j: (0, 0)),
            pl.BlockSpec((NSTACK, 3, HID), lambda b, j: (0, 0, 0)),
            pl.BlockSpec((NSTACK, HID), lambda b, j: (0, 0)),
            pl.BlockSpec((NSTACK, HID, HID), lambda b, j: (0, 0, 0)),
            pl.BlockSpec((NSTACK, HID), lambda b, j: (0, 0)),
            pl.BlockSpec((NSTACK, HID, 1), lambda b, j: (0, 0, 0)),
            pl.BlockSpec((NSTACK, 1), lambda b, j: (0, 0)),
        ],
        out_specs=pl.BlockSpec((B, TDIM * NNODE), lambda b, j: (0, 0)),
        out_shape=jax.ShapeDtypeStruct((B, TDIM * NNODE), jnp.float32),
    )(ft, bd, W0, b0, W1, b1, W2, b2)

    pooled = gin
    logits = pl.pallas_call(
        _mlp_body,
        out_shape=jax.ShapeDtypeStruct((B, 60), jnp.float32),
    )(pooled, Wf0, bf0.reshape(1, -1), Wf1, bf1.reshape(1, -1))
    return logits
